# baseline (speedup 1.0000x reference)
import functools

import jax
import jax.numpy as jnp
from jax import lax
from jax.experimental import pallas as pl
from jax.experimental.pallas import tpu as pltpu
from jax.experimental.pallas import tpu_sc as plsc

_N = 10000
_E = 320000
_F = 128
_H = 32
_Z = 16

_NC = 2
_NS = 16
_NW = _NC * _NS
_GRP = 128
_NG = 79
_EPAD = _NW * _NG * _GRP
_NPAD = 10240


def _sc_spmm(srcg, dstg, wg, dense, zeros):
    mesh = plsc.VectorSubcoreMesh(core_axis_name="c", subcore_axis_name="s",
                                  num_cores=_NC, num_subcores=_NS)
    rows_per_tile = _NPAD // _NS

    @functools.partial(
        pl.kernel,
        out_type=jax.ShapeDtypeStruct((_NC, _NPAD, _H), jnp.float32),
        mesh=mesh,
        scratch_types=[
            pltpu.VMEM((_NG, _GRP), jnp.int32),
            pltpu.VMEM((_NG, _GRP), jnp.int32),
            pltpu.VMEM((_NG, _GRP), jnp.float32),
            pltpu.VMEM((2, _GRP, _H), jnp.float32),
            pltpu.VMEM_SHARED((_NPAD, _H), jnp.float32),
            pltpu.SemaphoreType.DMA,
            pltpu.SemaphoreType.DMA,
        ],
        compiler_params=pltpu.CompilerParams(use_tc_tiling_on_sc=False),
    )
    def spmm_kernel(src_hbm, dst_hbm, w_hbm, dense_hbm, zero_hbm, out_hbm,
                    src_v, dst_v, w_v, rows_v, acc_sh, gsem, ssem):
        c = lax.axis_index("c")
        s = lax.axis_index("s")
        wid = s * _NC + c
        row0 = s * rows_per_tile

        pltpu.sync_copy(zero_hbm.at[pl.ds(row0, rows_per_tile)],
                        acc_sh.at[pl.ds(row0, rows_per_tile)])
        pltpu.sync_copy(src_hbm.at[wid], src_v)
        pltpu.sync_copy(dst_hbm.at[wid], dst_v)
        pltpu.sync_copy(w_hbm.at[wid], w_v)
        plsc.subcore_barrier()

        def wait_scatter():
            pltpu.make_async_copy(rows_v.at[0], acc_sh.at[dst_v.at[0]],
                                  ssem).wait()

        pltpu.async_copy(dense_hbm.at[src_v.at[0]], rows_v.at[0], gsem)

        def group_body(g, carry):
            slot = lax.rem(g, 2)
            nxt = lax.rem(g + 1, 2)

            @pl.when(g >= 1)
            def _():
                wait_scatter()

            @pl.when(g + 1 < _NG)
            def _():
                pltpu.async_copy(dense_hbm.at[src_v.at[g + 1]],
                                 rows_v.at[nxt], gsem)

            pltpu.make_async_copy(dense_hbm.at[src_v.at[g]],
                                  rows_v.at[slot], gsem).wait()

            def scale_body(k, carry2):
                w16 = w_v[g, pl.ds(k * 16, 16)]
                for l in range(16):
                    i = k * 16 + l
                    wsc = w16[l]
                    rows_v[slot, i, pl.ds(0, 16)] = (
                        rows_v[slot, i, pl.ds(0, 16)] * wsc)
                    rows_v[slot, i, pl.ds(16, 16)] = (
                        rows_v[slot, i, pl.ds(16, 16)] * wsc)
                return carry2

            lax.fori_loop(0, _GRP // 16, scale_body, 0)
            pltpu.async_copy(rows_v.at[slot], acc_sh.at[dst_v.at[g]], ssem,
                             add=True)
            return carry

        lax.fori_loop(0, _NG, group_body, 0)
        wait_scatter()

        plsc.subcore_barrier()
        pltpu.sync_copy(acc_sh.at[pl.ds(row0, rows_per_tile)],
                        out_hbm.at[c, pl.ds(row0, rows_per_tile)])

    return spmm_kernel(srcg, dstg, wg, dense, zeros)


def _xw_body(x_ref, w_ref, o_ref):
    o_ref[...] = jnp.dot(x_ref[...], w_ref[...],
                         preferred_element_type=jnp.float32)


def _tc_xw(x, w0):
    return pl.pallas_call(
        _xw_body,
        grid=(5,),
        in_specs=[
            pl.BlockSpec((2000, _F), lambda i: (i, 0)),
            pl.BlockSpec((_F, _H), lambda i: (0, 0)),
        ],
        out_specs=pl.BlockSpec((2000, _H), lambda i: (i, 0)),
        out_shape=jax.ShapeDtypeStruct((_N, _H), jnp.float32),
    )(x, w0)


def _combine_body(p_ref, o_ref):
    o_ref[...] = jnp.maximum(p_ref[0] + p_ref[1], 0.0)


def _tc_combine_relu(p):
    return pl.pallas_call(
        _combine_body,
        grid=(5,),
        in_specs=[pl.BlockSpec((_NC, 2000, _H), lambda i: (0, i, 0))],
        out_specs=pl.BlockSpec((2000, _H), lambda i: (i, 0)),
        out_shape=jax.ShapeDtypeStruct((_N, _H), jnp.float32),
    )(p)


def _heads_body(p_ref, wm_ref, ws_ref, eps_ref, m_ref, s_ref, z_ref):
    g = p_ref[0] + p_ref[1]
    m = jnp.dot(g, wm_ref[...], preferred_element_type=jnp.float32)
    st = jnp.dot(g, ws_ref[...], preferred_element_type=jnp.float32)
    m_ref[...] = m
    s_ref[...] = st
    z = m + eps_ref[...] * jnp.exp(st)
    z_ref[...] = z.astype(jnp.bfloat16)


def _tc_heads(p, w_mean, w_std, eps):
    return pl.pallas_call(
        _heads_body,
        grid=(5,),
        in_specs=[
            pl.BlockSpec((_NC, 2000, _H), lambda i: (0, i, 0)),
            pl.BlockSpec((_H, _Z), lambda i: (0, 0)),
            pl.BlockSpec((_H, _Z), lambda i: (0, 0)),
            pl.BlockSpec((2000, _Z), lambda i: (i, 0)),
        ],
        out_specs=[
            pl.BlockSpec((2000, _Z), lambda i: (i, 0)),
            pl.BlockSpec((2000, _Z), lambda i: (i, 0)),
            pl.BlockSpec((2000, _Z), lambda i: (i, 0)),
        ],
        out_shape=[
            jax.ShapeDtypeStruct((_N, _Z), jnp.float32),
            jax.ShapeDtypeStruct((_N, _Z), jnp.float32),
            jax.ShapeDtypeStruct((_N, _Z), jnp.bfloat16),
        ],
    )(p, w_mean, w_std, eps)


def _zzt_body(zr_ref, zc_ref, o_ref):
    o_ref[...] = lax.dot_general(
        zr_ref[...], zc_ref[...],
        dimension_numbers=(((1,), (1,)), ((), ())),
        preferred_element_type=jnp.float32,
    )


def _tc_zzt(zb):
    br, bc = 512, 2048
    return pl.pallas_call(
        _zzt_body,
        grid=(20, 5),
        in_specs=[
            pl.BlockSpec((br, _Z), lambda i, j: (i, 0)),
            pl.BlockSpec((bc, _Z), lambda i, j: (j, 0)),
        ],
        out_specs=pl.BlockSpec((br, bc), lambda i, j: (i, j)),
        out_shape=jax.ShapeDtypeStruct((_N, _N), jnp.float32),
        compiler_params=pltpu.CompilerParams(
            dimension_semantics=("arbitrary", "arbitrary"),
        ),
    )(zb, zb)


@jax.jit
def kernel(x, edge_index, edge_weight, W0, W_mean, W_std, eps):
    pad = _EPAD - _E
    src = jnp.concatenate([edge_index[0], jnp.zeros((pad,), jnp.int32)])
    dst = jnp.concatenate([edge_index[1], jnp.zeros((pad,), jnp.int32)])
    wpad = jnp.concatenate([edge_weight, jnp.zeros((pad,), jnp.float32)])
    srcg = src.reshape(_NW, _NG, _GRP)
    dstg = dst.reshape(_NW, _NG, _GRP)
    wg = wpad.reshape(_NW, _NG, _GRP)
    zeros = jnp.zeros((_NPAD, _H), jnp.float32)

    xw = _tc_xw(x, W0)
    p1 = _sc_spmm(srcg, dstg, wg, xw, zeros)
    h = _tc_combine_relu(p1)
    p2 = _sc_spmm(srcg, dstg, wg, h, zeros)
    means, stds, zb = _tc_heads(p2, W_mean, W_std, eps)
    recon = _tc_zzt(zb)
    return recon.reshape(-1), means, stds


# speedup vs baseline: 1.0216x; 1.0216x over previous
import functools

import jax
import jax.numpy as jnp
from jax import lax
from jax.experimental import pallas as pl
from jax.experimental.pallas import tpu as pltpu
from jax.experimental.pallas import tpu_sc as plsc

_N = 10000
_E = 320000
_F = 128
_H = 32
_Z = 16

_NC = 2
_NS = 16
_NW = _NC * _NS
_GRP = 128
_NG = 79
_EPAD = _NW * _NG * _GRP
_NPAD = 10240


def _sc_spmm(srcg, dstg, wg, dense, zeros):
    mesh = plsc.VectorSubcoreMesh(core_axis_name="c", subcore_axis_name="s",
                                  num_cores=_NC, num_subcores=_NS)
    rows_per_tile = _NPAD // _NS

    @functools.partial(
        pl.kernel,
        out_type=jax.ShapeDtypeStruct((_NC, _NPAD, _H), jnp.float32),
        mesh=mesh,
        scratch_types=[
            pltpu.VMEM((_NG, _GRP), jnp.int32),
            pltpu.VMEM((_NG, _GRP), jnp.int32),
            pltpu.VMEM((_NG, _GRP), jnp.float32),
            pltpu.VMEM((4, _GRP, _H), jnp.float32),
            pltpu.VMEM_SHARED((_NPAD, _H), jnp.float32),
            pltpu.SemaphoreType.DMA,
            pltpu.SemaphoreType.DMA,
        ],
        compiler_params=pltpu.CompilerParams(use_tc_tiling_on_sc=False),
    )
    def spmm_kernel(src_hbm, dst_hbm, w_hbm, dense_hbm, zero_hbm, out_hbm,
                    src_v, dst_v, w_v, rows_v, acc_sh, gsem, ssem):
        c = lax.axis_index("c")
        s = lax.axis_index("s")
        wid = s * _NC + c
        row0 = s * rows_per_tile

        pltpu.sync_copy(zero_hbm.at[pl.ds(row0, rows_per_tile)],
                        acc_sh.at[pl.ds(row0, rows_per_tile)])
        pltpu.sync_copy(src_hbm.at[wid], src_v)
        pltpu.sync_copy(dst_hbm.at[wid], dst_v)
        pltpu.sync_copy(w_hbm.at[wid], w_v)
        plsc.subcore_barrier()

        def wait_scatter():
            pltpu.make_async_copy(rows_v.at[0], acc_sh.at[dst_v.at[0]],
                                  ssem).wait()

        for p in range(3):
            pltpu.async_copy(dense_hbm.at[src_v.at[p]], rows_v.at[p], gsem)

        def group_body(g, carry):
            slot = lax.rem(g, 4)
            nxt = lax.rem(g + 3, 4)

            @pl.when(g >= 1)
            def _():
                wait_scatter()

            @pl.when(g + 3 < _NG)
            def _():
                pltpu.async_copy(dense_hbm.at[src_v.at[g + 3]],
                                 rows_v.at[nxt], gsem)

            pltpu.make_async_copy(dense_hbm.at[src_v.at[g]],
                                  rows_v.at[slot], gsem).wait()

            def scale_body(k, carry2):
                w16 = w_v[g, pl.ds(k * 16, 16)]
                for l in range(16):
                    i = k * 16 + l
                    wsc = w16[l]
                    rows_v[slot, i, pl.ds(0, 16)] = (
                        rows_v[slot, i, pl.ds(0, 16)] * wsc)
                    rows_v[slot, i, pl.ds(16, 16)] = (
                        rows_v[slot, i, pl.ds(16, 16)] * wsc)
                return carry2

            lax.fori_loop(0, _GRP // 16, scale_body, 0)
            pltpu.async_copy(rows_v.at[slot], acc_sh.at[dst_v.at[g]], ssem,
                             add=True)
            return carry

        lax.fori_loop(0, _NG, group_body, 0)
        wait_scatter()

        plsc.subcore_barrier()
        pltpu.sync_copy(acc_sh.at[pl.ds(row0, rows_per_tile)],
                        out_hbm.at[c, pl.ds(row0, rows_per_tile)])

    return spmm_kernel(srcg, dstg, wg, dense, zeros)


def _xw_body(x_ref, w_ref, o_ref):
    o_ref[...] = jnp.dot(x_ref[...], w_ref[...],
                         preferred_element_type=jnp.float32)


def _tc_xw(x, w0):
    return pl.pallas_call(
        _xw_body,
        grid=(5,),
        in_specs=[
            pl.BlockSpec((2000, _F), lambda i: (i, 0)),
            pl.BlockSpec((_F, _H), lambda i: (0, 0)),
        ],
        out_specs=pl.BlockSpec((2000, _H), lambda i: (i, 0)),
        out_shape=jax.ShapeDtypeStruct((_N, _H), jnp.float32),
    )(x, w0)


def _combine_body(p_ref, o_ref):
    o_ref[...] = jnp.maximum(p_ref[0] + p_ref[1], 0.0)


def _tc_combine_relu(p):
    return pl.pallas_call(
        _combine_body,
        grid=(5,),
        in_specs=[pl.BlockSpec((_NC, 2000, _H), lambda i: (0, i, 0))],
        out_specs=pl.BlockSpec((2000, _H), lambda i: (i, 0)),
        out_shape=jax.ShapeDtypeStruct((_N, _H), jnp.float32),
    )(p)


def _heads_body(p_ref, wm_ref, ws_ref, eps_ref, m_ref, s_ref, z_ref):
    g = p_ref[0] + p_ref[1]
    m = jnp.dot(g, wm_ref[...], preferred_element_type=jnp.float32)
    st = jnp.dot(g, ws_ref[...], preferred_element_type=jnp.float32)
    m_ref[...] = m
    s_ref[...] = st
    z = m + eps_ref[...] * jnp.exp(st)
    z_ref[...] = z.astype(jnp.bfloat16)


def _tc_heads(p, w_mean, w_std, eps):
    return pl.pallas_call(
        _heads_body,
        grid=(5,),
        in_specs=[
            pl.BlockSpec((_NC, 2000, _H), lambda i: (0, i, 0)),
            pl.BlockSpec((_H, _Z), lambda i: (0, 0)),
            pl.BlockSpec((_H, _Z), lambda i: (0, 0)),
            pl.BlockSpec((2000, _Z), lambda i: (i, 0)),
        ],
        out_specs=[
            pl.BlockSpec((2000, _Z), lambda i: (i, 0)),
            pl.BlockSpec((2000, _Z), lambda i: (i, 0)),
            pl.BlockSpec((2000, _Z), lambda i: (i, 0)),
        ],
        out_shape=[
            jax.ShapeDtypeStruct((_N, _Z), jnp.float32),
            jax.ShapeDtypeStruct((_N, _Z), jnp.float32),
            jax.ShapeDtypeStruct((_N, _Z), jnp.bfloat16),
        ],
    )(p, w_mean, w_std, eps)


def _zzt_body(zr_ref, zc_ref, o_ref):
    o_ref[...] = lax.dot_general(
        zr_ref[...], zc_ref[...],
        dimension_numbers=(((1,), (1,)), ((), ())),
        preferred_element_type=jnp.float32,
    )


def _tc_zzt(zb):
    br, bc = 512, 2048
    return pl.pallas_call(
        _zzt_body,
        grid=(20, 5),
        in_specs=[
            pl.BlockSpec((br, _Z), lambda i, j: (i, 0)),
            pl.BlockSpec((bc, _Z), lambda i, j: (j, 0)),
        ],
        out_specs=pl.BlockSpec((br, bc), lambda i, j: (i, j)),
        out_shape=jax.ShapeDtypeStruct((_N, _N), jnp.float32),
        compiler_params=pltpu.CompilerParams(
            dimension_semantics=("arbitrary", "arbitrary"),
        ),
    )(zb, zb)


@jax.jit
def kernel(x, edge_index, edge_weight, W0, W_mean, W_std, eps):
    pad = _EPAD - _E
    src = jnp.concatenate([edge_index[0], jnp.zeros((pad,), jnp.int32)])
    dst = jnp.concatenate([edge_index[1], jnp.zeros((pad,), jnp.int32)])
    wpad = jnp.concatenate([edge_weight, jnp.zeros((pad,), jnp.float32)])
    srcg = src.reshape(_NW, _NG, _GRP)
    dstg = dst.reshape(_NW, _NG, _GRP)
    wg = wpad.reshape(_NW, _NG, _GRP)
    zeros = jnp.zeros((_NPAD, _H), jnp.float32)

    xw = _tc_xw(x, W0)
    p1 = _sc_spmm(srcg, dstg, wg, xw, zeros)
    h = _tc_combine_relu(p1)
    p2 = _sc_spmm(srcg, dstg, wg, h, zeros)
    means, stds, zb = _tc_heads(p2, W_mean, W_std, eps)
    recon = _tc_zzt(zb)
    return recon.reshape(-1), means, stds


# speedup vs baseline: 1.0216x; 1.0001x over previous
import functools

import jax
import jax.numpy as jnp
from jax import lax
from jax.experimental import pallas as pl
from jax.experimental.pallas import tpu as pltpu
from jax.experimental.pallas import tpu_sc as plsc

_N = 10000
_E = 320000
_F = 128
_H = 32
_Z = 16

_NC = 2
_NS = 16
_NW = _NC * _NS
_GRP = 128
_NG = 79
_EPAD = _NW * _NG * _GRP
_NPAD = 10240


def _sc_spmm(srcg, dstg, wg, dense, zeros):
    mesh = plsc.VectorSubcoreMesh(core_axis_name="c", subcore_axis_name="s",
                                  num_cores=_NC, num_subcores=_NS)
    rows_per_tile = _NPAD // _NS

    @functools.partial(
        pl.kernel,
        out_type=jax.ShapeDtypeStruct((_NC, _NPAD, _H), jnp.float32),
        mesh=mesh,
        scratch_types=[
            pltpu.VMEM((_NG, _GRP), jnp.int32),
            pltpu.VMEM((_NG, _GRP), jnp.int32),
            pltpu.VMEM((_NG, _GRP), jnp.float32),
            pltpu.VMEM((4, _GRP, _H), jnp.float32),
            pltpu.VMEM_SHARED((_NPAD, _H), jnp.float32),
            pltpu.SemaphoreType.DMA,
            pltpu.SemaphoreType.DMA,
        ],
        compiler_params=pltpu.CompilerParams(use_tc_tiling_on_sc=False),
    )
    def spmm_kernel(src_hbm, dst_hbm, w_hbm, dense_hbm, zero_hbm, out_hbm,
                    src_v, dst_v, w_v, rows_v, acc_sh, gsem, ssem):
        c = lax.axis_index("c")
        s = lax.axis_index("s")
        wid = s * _NC + c
        row0 = s * rows_per_tile

        pltpu.sync_copy(zero_hbm.at[pl.ds(row0, rows_per_tile)],
                        acc_sh.at[pl.ds(row0, rows_per_tile)])
        pltpu.sync_copy(src_hbm.at[wid], src_v)
        pltpu.sync_copy(dst_hbm.at[wid], dst_v)
        pltpu.sync_copy(w_hbm.at[wid], w_v)
        plsc.subcore_barrier()

        def wait_scatter():
            pltpu.make_async_copy(rows_v.at[0], acc_sh.at[dst_v.at[0]],
                                  ssem).wait()

        for p in range(3):
            pltpu.async_copy(dense_hbm.at[src_v.at[p]], rows_v.at[p], gsem)

        def group_body(g, carry):
            slot = lax.rem(g, 4)
            nxt = lax.rem(g + 3, 4)

            @pl.when(g >= 1)
            def _():
                wait_scatter()

            @pl.when(g + 3 < _NG)
            def _():
                pltpu.async_copy(dense_hbm.at[src_v.at[g + 3]],
                                 rows_v.at[nxt], gsem)

            pltpu.make_async_copy(dense_hbm.at[src_v.at[g]],
                                  rows_v.at[slot], gsem).wait()

            for k in range(_GRP // 16):
                w16 = w_v[g, pl.ds(k * 16, 16)]
                for l in range(16):
                    i = k * 16 + l
                    wsc = w16[l]
                    rows_v[slot, i, pl.ds(0, 16)] = (
                        rows_v[slot, i, pl.ds(0, 16)] * wsc)
                    rows_v[slot, i, pl.ds(16, 16)] = (
                        rows_v[slot, i, pl.ds(16, 16)] * wsc)
            pltpu.async_copy(rows_v.at[slot], acc_sh.at[dst_v.at[g]], ssem,
                             add=True)
            return carry

        lax.fori_loop(0, _NG, group_body, 0)
        wait_scatter()

        plsc.subcore_barrier()
        pltpu.sync_copy(acc_sh.at[pl.ds(row0, rows_per_tile)],
                        out_hbm.at[c, pl.ds(row0, rows_per_tile)])

    return spmm_kernel(srcg, dstg, wg, dense, zeros)


def _xw_body(x_ref, w_ref, o_ref):
    o_ref[...] = jnp.dot(x_ref[...], w_ref[...],
                         preferred_element_type=jnp.float32)


def _tc_xw(x, w0):
    return pl.pallas_call(
        _xw_body,
        grid=(5,),
        in_specs=[
            pl.BlockSpec((2000, _F), lambda i: (i, 0)),
            pl.BlockSpec((_F, _H), lambda i: (0, 0)),
        ],
        out_specs=pl.BlockSpec((2000, _H), lambda i: (i, 0)),
        out_shape=jax.ShapeDtypeStruct((_N, _H), jnp.float32),
    )(x, w0)


def _combine_body(p_ref, o_ref):
    o_ref[...] = jnp.maximum(p_ref[0] + p_ref[1], 0.0)


def _tc_combine_relu(p):
    return pl.pallas_call(
        _combine_body,
        grid=(5,),
        in_specs=[pl.BlockSpec((_NC, 2000, _H), lambda i: (0, i, 0))],
        out_specs=pl.BlockSpec((2000, _H), lambda i: (i, 0)),
        out_shape=jax.ShapeDtypeStruct((_N, _H), jnp.float32),
    )(p)


def _heads_body(p_ref, wm_ref, ws_ref, eps_ref, m_ref, s_ref, z_ref):
    g = p_ref[0] + p_ref[1]
    m = jnp.dot(g, wm_ref[...], preferred_element_type=jnp.float32)
    st = jnp.dot(g, ws_ref[...], preferred_element_type=jnp.float32)
    m_ref[...] = m
    s_ref[...] = st
    z = m + eps_ref[...] * jnp.exp(st)
    z_ref[...] = z.astype(jnp.bfloat16)


def _tc_heads(p, w_mean, w_std, eps):
    return pl.pallas_call(
        _heads_body,
        grid=(5,),
        in_specs=[
            pl.BlockSpec((_NC, 2000, _H), lambda i: (0, i, 0)),
            pl.BlockSpec((_H, _Z), lambda i: (0, 0)),
            pl.BlockSpec((_H, _Z), lambda i: (0, 0)),
            pl.BlockSpec((2000, _Z), lambda i: (i, 0)),
        ],
        out_specs=[
            pl.BlockSpec((2000, _Z), lambda i: (i, 0)),
            pl.BlockSpec((2000, _Z), lambda i: (i, 0)),
            pl.BlockSpec((2000, _Z), lambda i: (i, 0)),
        ],
        out_shape=[
            jax.ShapeDtypeStruct((_N, _Z), jnp.float32),
            jax.ShapeDtypeStruct((_N, _Z), jnp.float32),
            jax.ShapeDtypeStruct((_N, _Z), jnp.bfloat16),
        ],
    )(p, w_mean, w_std, eps)


def _zzt_body(zr_ref, zc_ref, o_ref):
    o_ref[...] = lax.dot_general(
        zr_ref[...], zc_ref[...],
        dimension_numbers=(((1,), (1,)), ((), ())),
        preferred_element_type=jnp.float32,
    )


def _tc_zzt(zb):
    br, bc = 512, 2048
    return pl.pallas_call(
        _zzt_body,
        grid=(20, 5),
        in_specs=[
            pl.BlockSpec((br, _Z), lambda i, j: (i, 0)),
            pl.BlockSpec((bc, _Z), lambda i, j: (j, 0)),
        ],
        out_specs=pl.BlockSpec((br, bc), lambda i, j: (i, j)),
        out_shape=jax.ShapeDtypeStruct((_N, _N), jnp.float32),
        compiler_params=pltpu.CompilerParams(
            dimension_semantics=("arbitrary", "arbitrary"),
        ),
    )(zb, zb)


@jax.jit
def kernel(x, edge_index, edge_weight, W0, W_mean, W_std, eps):
    pad = _EPAD - _E
    src = jnp.concatenate([edge_index[0], jnp.zeros((pad,), jnp.int32)])
    dst = jnp.concatenate([edge_index[1], jnp.zeros((pad,), jnp.int32)])
    wpad = jnp.concatenate([edge_weight, jnp.zeros((pad,), jnp.float32)])
    srcg = src.reshape(_NW, _NG, _GRP)
    dstg = dst.reshape(_NW, _NG, _GRP)
    wg = wpad.reshape(_NW, _NG, _GRP)
    zeros = jnp.zeros((_NPAD, _H), jnp.float32)

    xw = _tc_xw(x, W0)
    p1 = _sc_spmm(srcg, dstg, wg, xw, zeros)
    h = _tc_combine_relu(p1)
    p2 = _sc_spmm(srcg, dstg, wg, h, zeros)
    means, stds, zb = _tc_heads(p2, W_mean, W_std, eps)
    recon = _tc_zzt(zb)
    return recon.reshape(-1), means, stds


# speedup vs baseline: 1.0359x; 1.0139x over previous
import functools

import jax
import jax.numpy as jnp
from jax import lax
from jax.experimental import pallas as pl
from jax.experimental.pallas import tpu as pltpu
from jax.experimental.pallas import tpu_sc as plsc

_N = 10000
_E = 320000
_F = 128
_H = 32
_Z = 16

_NC = 2
_NS = 16
_NW = _NC * _NS
_GRP = 128
_NG = 79
_EPAD = _NW * _NG * _GRP
_TG = _NW * _NG
_NG_C0 = 94
_NG_C1 = 2 * _NG - _NG_C0
_NPAD = 10240


def _sc_spmm(srcg, dstg, wg, dense, zeros):
    mesh = plsc.VectorSubcoreMesh(core_axis_name="c", subcore_axis_name="s",
                                  num_cores=_NC, num_subcores=_NS)
    rows_per_tile = _NPAD // _NS

    @functools.partial(
        pl.kernel,
        out_type=jax.ShapeDtypeStruct((_NC, _NPAD, _H), jnp.float32),
        mesh=mesh,
        scratch_types=[
            pltpu.VMEM((_NG_C0, _GRP), jnp.int32),
            pltpu.VMEM((_NG_C0, _GRP), jnp.int32),
            pltpu.VMEM((_NG_C0, _GRP), jnp.float32),
            pltpu.VMEM((4, _GRP, _H), jnp.float32),
            pltpu.VMEM_SHARED((_NPAD, _H), jnp.float32),
            pltpu.SemaphoreType.DMA,
            pltpu.SemaphoreType.DMA,
        ],
        compiler_params=pltpu.CompilerParams(use_tc_tiling_on_sc=False),
    )
    def spmm_kernel(src_hbm, dst_hbm, w_hbm, dense_hbm, zero_hbm, out_hbm,
                    src_v, dst_v, w_v, rows_v, acc_sh, gsem, ssem):
        c = lax.axis_index("c")
        s = lax.axis_index("s")
        row0 = s * rows_per_tile

        pltpu.sync_copy(zero_hbm.at[pl.ds(row0, rows_per_tile)],
                        acc_sh.at[pl.ds(row0, rows_per_tile)])

        def wait_scatter():
            pltpu.make_async_copy(rows_v.at[0], acc_sh.at[dst_v.at[0]],
                                  ssem).wait()

        def run_half(ng, start):
            pltpu.sync_copy(src_hbm.at[pl.ds(start, ng)],
                            src_v.at[pl.ds(0, ng)])
            pltpu.sync_copy(dst_hbm.at[pl.ds(start, ng)],
                            dst_v.at[pl.ds(0, ng)])
            pltpu.sync_copy(w_hbm.at[pl.ds(start, ng)],
                            w_v.at[pl.ds(0, ng)])
            plsc.subcore_barrier()

            for p in range(3):
                pltpu.async_copy(dense_hbm.at[src_v.at[p]], rows_v.at[p],
                                 gsem)

            def group_body(g, carry):
                slot = lax.rem(g, 4)
                nxt = lax.rem(g + 3, 4)

                @pl.when(g >= 1)
                def _():
                    wait_scatter()

                @pl.when(g + 3 < ng)
                def _():
                    pltpu.async_copy(dense_hbm.at[src_v.at[g + 3]],
                                     rows_v.at[nxt], gsem)

                pltpu.make_async_copy(dense_hbm.at[src_v.at[g]],
                                      rows_v.at[slot], gsem).wait()

                for k in range(_GRP // 16):
                    w16 = w_v[g, pl.ds(k * 16, 16)]
                    for l in range(16):
                        i = k * 16 + l
                        wsc = w16[l]
                        rows_v[slot, i, pl.ds(0, 16)] = (
                            rows_v[slot, i, pl.ds(0, 16)] * wsc)
                        rows_v[slot, i, pl.ds(16, 16)] = (
                            rows_v[slot, i, pl.ds(16, 16)] * wsc)
                pltpu.async_copy(rows_v.at[slot], acc_sh.at[dst_v.at[g]],
                                 ssem, add=True)
                return carry

            lax.fori_loop(0, ng, group_body, 0)
            wait_scatter()

        @pl.when(c == 0)
        def _():
            run_half(_NG_C0, s * _NG_C0)

        @pl.when(c == 1)
        def _():
            run_half(_NG_C1, _NS * _NG_C0 + s * _NG_C1)

        plsc.subcore_barrier()
        pltpu.sync_copy(acc_sh.at[pl.ds(row0, rows_per_tile)],
                        out_hbm.at[c, pl.ds(row0, rows_per_tile)])

    return spmm_kernel(srcg, dstg, wg, dense, zeros)


def _xw_body(x_ref, w_ref, o_ref):
    o_ref[...] = jnp.dot(x_ref[...], w_ref[...],
                         preferred_element_type=jnp.float32)


def _tc_xw(x, w0):
    return pl.pallas_call(
        _xw_body,
        grid=(5,),
        in_specs=[
            pl.BlockSpec((2000, _F), lambda i: (i, 0)),
            pl.BlockSpec((_F, _H), lambda i: (0, 0)),
        ],
        out_specs=pl.BlockSpec((2000, _H), lambda i: (i, 0)),
        out_shape=jax.ShapeDtypeStruct((_N, _H), jnp.float32),
    )(x, w0)


def _combine_body(p_ref, o_ref):
    o_ref[...] = jnp.maximum(p_ref[0] + p_ref[1], 0.0)


def _tc_combine_relu(p):
    return pl.pallas_call(
        _combine_body,
        grid=(5,),
        in_specs=[pl.BlockSpec((_NC, 2000, _H), lambda i: (0, i, 0))],
        out_specs=pl.BlockSpec((2000, _H), lambda i: (i, 0)),
        out_shape=jax.ShapeDtypeStruct((_N, _H), jnp.float32),
    )(p)


def _heads_body(p_ref, wm_ref, ws_ref, eps_ref, m_ref, s_ref, z_ref):
    g = p_ref[0] + p_ref[1]
    m = jnp.dot(g, wm_ref[...], preferred_element_type=jnp.float32)
    st = jnp.dot(g, ws_ref[...], preferred_element_type=jnp.float32)
    m_ref[...] = m
    s_ref[...] = st
    z = m + eps_ref[...] * jnp.exp(st)
    z_ref[...] = z.astype(jnp.bfloat16)


def _tc_heads(p, w_mean, w_std, eps):
    return pl.pallas_call(
        _heads_body,
        grid=(5,),
        in_specs=[
            pl.BlockSpec((_NC, 2000, _H), lambda i: (0, i, 0)),
            pl.BlockSpec((_H, _Z), lambda i: (0, 0)),
            pl.BlockSpec((_H, _Z), lambda i: (0, 0)),
            pl.BlockSpec((2000, _Z), lambda i: (i, 0)),
        ],
        out_specs=[
            pl.BlockSpec((2000, _Z), lambda i: (i, 0)),
            pl.BlockSpec((2000, _Z), lambda i: (i, 0)),
            pl.BlockSpec((2000, _Z), lambda i: (i, 0)),
        ],
        out_shape=[
            jax.ShapeDtypeStruct((_N, _Z), jnp.float32),
            jax.ShapeDtypeStruct((_N, _Z), jnp.float32),
            jax.ShapeDtypeStruct((_N, _Z), jnp.bfloat16),
        ],
    )(p, w_mean, w_std, eps)


def _zzt_body(zr_ref, zc_ref, o_ref):
    o_ref[...] = lax.dot_general(
        zr_ref[...], zc_ref[...],
        dimension_numbers=(((1,), (1,)), ((), ())),
        preferred_element_type=jnp.float32,
    )


def _tc_zzt(zb):
    br, bc = 512, 2048
    return pl.pallas_call(
        _zzt_body,
        grid=(20, 5),
        in_specs=[
            pl.BlockSpec((br, _Z), lambda i, j: (i, 0)),
            pl.BlockSpec((bc, _Z), lambda i, j: (j, 0)),
        ],
        out_specs=pl.BlockSpec((br, bc), lambda i, j: (i, j)),
        out_shape=jax.ShapeDtypeStruct((_N, _N), jnp.float32),
        compiler_params=pltpu.CompilerParams(
            dimension_semantics=("arbitrary", "arbitrary"),
        ),
    )(zb, zb)


@jax.jit
def kernel(x, edge_index, edge_weight, W0, W_mean, W_std, eps):
    pad = _EPAD - _E
    src = jnp.concatenate([edge_index[0], jnp.zeros((pad,), jnp.int32)])
    dst = jnp.concatenate([edge_index[1], jnp.zeros((pad,), jnp.int32)])
    wpad = jnp.concatenate([edge_weight, jnp.zeros((pad,), jnp.float32)])
    srcg = src.reshape(_TG, _GRP)
    dstg = dst.reshape(_TG, _GRP)
    wg = wpad.reshape(_TG, _GRP)
    zeros = jnp.zeros((_NPAD, _H), jnp.float32)

    xw = _tc_xw(x, W0)
    p1 = _sc_spmm(srcg, dstg, wg, xw, zeros)
    h = _tc_combine_relu(p1)
    p2 = _sc_spmm(srcg, dstg, wg, h, zeros)
    means, stds, zb = _tc_heads(p2, W_mean, W_std, eps)
    recon = _tc_zzt(zb)
    return recon.reshape(-1), means, stds


# speedup vs baseline: 1.0437x; 1.0075x over previous
import functools

import jax
import jax.numpy as jnp
from jax import lax
from jax.experimental import pallas as pl
from jax.experimental.pallas import tpu as pltpu
from jax.experimental.pallas import tpu_sc as plsc

_N = 10000
_E = 320000
_F = 128
_H = 32
_Z = 16

_NC = 2
_NS = 16
_NW = _NC * _NS
_GRP = 128
_NG = 79
_EPAD = _NW * _NG * _GRP
_TG = _NW * _NG
_NG_C0 = 100
_NG_C1 = 2 * _NG - _NG_C0
_NPAD = 10240


def _sc_spmm(srcg, dstg, wg, dense, zeros):
    mesh = plsc.VectorSubcoreMesh(core_axis_name="c", subcore_axis_name="s",
                                  num_cores=_NC, num_subcores=_NS)
    rows_per_tile = _NPAD // _NS

    @functools.partial(
        pl.kernel,
        out_type=jax.ShapeDtypeStruct((_NC, _NPAD, _H), jnp.float32),
        mesh=mesh,
        scratch_types=[
            pltpu.VMEM((_NG_C0, _GRP), jnp.int32),
            pltpu.VMEM((_NG_C0, _GRP), jnp.int32),
            pltpu.VMEM((_NG_C0, _GRP), jnp.float32),
            pltpu.VMEM((4, _GRP, _H), jnp.float32),
            pltpu.VMEM_SHARED((_NPAD, _H), jnp.float32),
            pltpu.SemaphoreType.DMA,
            pltpu.SemaphoreType.DMA,
        ],
        compiler_params=pltpu.CompilerParams(use_tc_tiling_on_sc=False),
    )
    def spmm_kernel(src_hbm, dst_hbm, w_hbm, dense_hbm, zero_hbm, out_hbm,
                    src_v, dst_v, w_v, rows_v, acc_sh, gsem, ssem):
        c = lax.axis_index("c")
        s = lax.axis_index("s")
        row0 = s * rows_per_tile

        pltpu.sync_copy(zero_hbm.at[pl.ds(row0, rows_per_tile)],
                        acc_sh.at[pl.ds(row0, rows_per_tile)])

        def wait_scatter():
            pltpu.make_async_copy(rows_v.at[0], acc_sh.at[dst_v.at[0]],
                                  ssem).wait()

        def run_half(ng, start):
            pltpu.sync_copy(src_hbm.at[pl.ds(start, ng)],
                            src_v.at[pl.ds(0, ng)])
            pltpu.sync_copy(dst_hbm.at[pl.ds(start, ng)],
                            dst_v.at[pl.ds(0, ng)])
            pltpu.sync_copy(w_hbm.at[pl.ds(start, ng)],
                            w_v.at[pl.ds(0, ng)])
            plsc.subcore_barrier()

            for p in range(3):
                pltpu.async_copy(dense_hbm.at[src_v.at[p]], rows_v.at[p],
                                 gsem)

            def group_body(g, carry):
                slot = lax.rem(g, 4)
                nxt = lax.rem(g + 3, 4)

                @pl.when(g >= 1)
                def _():
                    wait_scatter()

                @pl.when(g + 3 < ng)
                def _():
                    pltpu.async_copy(dense_hbm.at[src_v.at[g + 3]],
                                     rows_v.at[nxt], gsem)

                pltpu.make_async_copy(dense_hbm.at[src_v.at[g]],
                                      rows_v.at[slot], gsem).wait()

                for k in range(_GRP // 16):
                    w16 = w_v[g, pl.ds(k * 16, 16)]
                    for l in range(16):
                        i = k * 16 + l
                        wsc = w16[l]
                        rows_v[slot, i, pl.ds(0, 16)] = (
                            rows_v[slot, i, pl.ds(0, 16)] * wsc)
                        rows_v[slot, i, pl.ds(16, 16)] = (
                            rows_v[slot, i, pl.ds(16, 16)] * wsc)
                pltpu.async_copy(rows_v.at[slot], acc_sh.at[dst_v.at[g]],
                                 ssem, add=True)
                return carry

            lax.fori_loop(0, ng, group_body, 0)
            wait_scatter()

        @pl.when(c == 0)
        def _():
            run_half(_NG_C0, s * _NG_C0)

        @pl.when(c == 1)
        def _():
            run_half(_NG_C1, _NS * _NG_C0 + s * _NG_C1)

        plsc.subcore_barrier()
        pltpu.sync_copy(acc_sh.at[pl.ds(row0, rows_per_tile)],
                        out_hbm.at[c, pl.ds(row0, rows_per_tile)])

    return spmm_kernel(srcg, dstg, wg, dense, zeros)


def _xw_body(x_ref, w_ref, o_ref):
    o_ref[...] = jnp.dot(x_ref[...], w_ref[...],
                         preferred_element_type=jnp.float32)


def _tc_xw(x, w0):
    return pl.pallas_call(
        _xw_body,
        grid=(5,),
        in_specs=[
            pl.BlockSpec((2000, _F), lambda i: (i, 0)),
            pl.BlockSpec((_F, _H), lambda i: (0, 0)),
        ],
        out_specs=pl.BlockSpec((2000, _H), lambda i: (i, 0)),
        out_shape=jax.ShapeDtypeStruct((_N, _H), jnp.float32),
    )(x, w0)


def _combine_body(p_ref, o_ref):
    o_ref[...] = jnp.maximum(p_ref[0] + p_ref[1], 0.0)


def _tc_combine_relu(p):
    return pl.pallas_call(
        _combine_body,
        grid=(5,),
        in_specs=[pl.BlockSpec((_NC, 2000, _H), lambda i: (0, i, 0))],
        out_specs=pl.BlockSpec((2000, _H), lambda i: (i, 0)),
        out_shape=jax.ShapeDtypeStruct((_N, _H), jnp.float32),
    )(p)


def _heads_body(p_ref, wm_ref, ws_ref, eps_ref, m_ref, s_ref, z_ref):
    g = p_ref[0] + p_ref[1]
    m = jnp.dot(g, wm_ref[...], preferred_element_type=jnp.float32)
    st = jnp.dot(g, ws_ref[...], preferred_element_type=jnp.float32)
    m_ref[...] = m
    s_ref[...] = st
    z = m + eps_ref[...] * jnp.exp(st)
    z_ref[...] = z.astype(jnp.bfloat16)


def _tc_heads(p, w_mean, w_std, eps):
    return pl.pallas_call(
        _heads_body,
        grid=(5,),
        in_specs=[
            pl.BlockSpec((_NC, 2000, _H), lambda i: (0, i, 0)),
            pl.BlockSpec((_H, _Z), lambda i: (0, 0)),
            pl.BlockSpec((_H, _Z), lambda i: (0, 0)),
            pl.BlockSpec((2000, _Z), lambda i: (i, 0)),
        ],
        out_specs=[
            pl.BlockSpec((2000, _Z), lambda i: (i, 0)),
            pl.BlockSpec((2000, _Z), lambda i: (i, 0)),
            pl.BlockSpec((2000, _Z), lambda i: (i, 0)),
        ],
        out_shape=[
            jax.ShapeDtypeStruct((_N, _Z), jnp.float32),
            jax.ShapeDtypeStruct((_N, _Z), jnp.float32),
            jax.ShapeDtypeStruct((_N, _Z), jnp.bfloat16),
        ],
    )(p, w_mean, w_std, eps)


def _zzt_body(zr_ref, zc_ref, o_ref):
    o_ref[...] = lax.dot_general(
        zr_ref[...], zc_ref[...],
        dimension_numbers=(((1,), (1,)), ((), ())),
        preferred_element_type=jnp.float32,
    )


def _tc_zzt(zb):
    br, bc = 512, 2048
    return pl.pallas_call(
        _zzt_body,
        grid=(20, 5),
        in_specs=[
            pl.BlockSpec((br, _Z), lambda i, j: (i, 0)),
            pl.BlockSpec((bc, _Z), lambda i, j: (j, 0)),
        ],
        out_specs=pl.BlockSpec((br, bc), lambda i, j: (i, j)),
        out_shape=jax.ShapeDtypeStruct((_N, _N), jnp.float32),
        compiler_params=pltpu.CompilerParams(
            dimension_semantics=("arbitrary", "arbitrary"),
        ),
    )(zb, zb)


@jax.jit
def kernel(x, edge_index, edge_weight, W0, W_mean, W_std, eps):
    pad = _EPAD - _E
    src = jnp.concatenate([edge_index[0], jnp.zeros((pad,), jnp.int32)])
    dst = jnp.concatenate([edge_index[1], jnp.zeros((pad,), jnp.int32)])
    wpad = jnp.concatenate([edge_weight, jnp.zeros((pad,), jnp.float32)])
    srcg = src.reshape(_TG, _GRP)
    dstg = dst.reshape(_TG, _GRP)
    wg = wpad.reshape(_TG, _GRP)
    zeros = jnp.zeros((_NPAD, _H), jnp.float32)

    xw = _tc_xw(x, W0)
    p1 = _sc_spmm(srcg, dstg, wg, xw, zeros)
    h = _tc_combine_relu(p1)
    p2 = _sc_spmm(srcg, dstg, wg, h, zeros)
    means, stds, zb = _tc_heads(p2, W_mean, W_std, eps)
    recon = _tc_zzt(zb)
    return recon.reshape(-1), means, stds
